# Initial kernel scaffold; baseline (speedup 1.0000x reference)
import functools

import jax
import jax.numpy as jnp
from jax import lax
from jax.experimental import pallas as pl
from jax.experimental.pallas import tpu as pltpu
from jax.experimental.pallas import tpu_sc as plsc

H, W, C = 200, 304, 256
OUT = 7
SCALE = 0.25
SNUM = 2
NPTS = OUT * SNUM
NROI = 5000
NW = 32
PER_W = 157
NPAD = NW * PER_W


def _body(table, rois, out, roi_v, itbl, ftbl, wtbl, idxb, rows, obuf, sem):
    wid = lax.axis_index("s") * 2 + lax.axis_index("c")
    base = wid * PER_W
    pltpu.sync_copy(rois.at[pl.ds(base, PER_W)], roi_v)

    lane = lax.iota(jnp.int32, 16)
    pv = (lane // 2).astype(jnp.float32)
    iv = (lane % 2).astype(jnp.float32)
    lanemask = lane < NPTS

    def roi_body(n, carry):
        b = roi_v[n, 0].astype(jnp.int32)
        bb = b * (H * W)
        sw = roi_v[n, 1] * SCALE
        sh = roi_v[n, 2] * SCALE
        ew = roi_v[n, 3] * SCALE
        eh = roi_v[n, 4] * SCALE
        bw = jnp.maximum(ew - sw, 1.0) * (1.0 / OUT)
        bh = jnp.maximum(eh - sh, 1.0) * (1.0 / OUT)
        xs = sw + pv * bw + (iv + 0.5) * (bw * (1.0 / SNUM))
        ys = sh + pv * bh + (iv + 0.5) * (bh * (1.0 / SNUM))
        vx = (xs > -1.0) & (xs < float(W)) & lanemask
        vy = (ys > -1.0) & (ys < float(H)) & lanemask
        xc = jnp.maximum(xs, 0.0)
        yc = jnp.maximum(ys, 0.0)
        x0 = jnp.minimum(xc.astype(jnp.int32), W - 1)
        y0 = jnp.minimum(yc.astype(jnp.int32), H - 1)
        x1 = jnp.minimum(x0 + 1, W - 1)
        y1 = jnp.minimum(y0 + 1, H - 1)
        lx = xc - x0.astype(jnp.float32)
        ly = yc - y0.astype(jnp.float32)
        zero = jnp.zeros_like(lx)
        hxv = jnp.where(vx, (1.0 - lx) * 0.5, zero)
        lxv = jnp.where(vx, lx * 0.5, zero)
        hyv = jnp.where(vy, (1.0 - ly) * 0.5, zero)
        lyv = jnp.where(vy, ly * 0.5, zero)
        itbl[0, :] = bb + y0 * W
        itbl[1, :] = bb + y1 * W
        itbl[2, :] = x0
        itbl[3, :] = x1
        ftbl[0, :] = hyv
        ftbl[1, :] = lyv
        ftbl[2, :] = hxv
        ftbl[3, :] = lxv

        for py in range(OUT):
            x0v = itbl[2, :]
            x1v = itbl[3, :]
            hxr = ftbl[2, :]
            lxr = ftbl[3, :]
            for si in range(2):
                i = py * 2 + si
                for a in range(2):
                    g2 = si * 2 + a
                    rb = itbl[a, i]
                    idxb[pl.ds((g2 * 2) * 16, 16)] = rb + x0v
                    idxb[pl.ds((g2 * 2 + 1) * 16, 16)] = rb + x1v
                    wy = ftbl[a, i]
                    wtbl[g2 * 2, :] = wy * hxr
                    wtbl[g2 * 2 + 1, :] = wy * lxr
            pltpu.async_copy(table.at[idxb], rows, sem).wait()

            def px_body(px, _):
                def cc_body(cc, _):
                    co = cc * 16
                    acc = jnp.zeros((16,), jnp.float32)
                    for g in range(8):
                        for sj in range(2):
                            j = px * 2 + sj
                            wsc = wtbl[g, j]
                            acc = acc + wsc * rows[g * 16 + j, pl.ds(co, 16)]
                    obuf[py * OUT + px, pl.ds(co, 16)] = acc
                    return 0

                return lax.fori_loop(0, C // 16, cc_body, 0)

            lax.fori_loop(0, OUT, px_body, 0)

        pltpu.sync_copy(obuf, out.at[base + n])
        return carry

    lax.fori_loop(0, PER_W, roi_body, 0)


def kernel(features, rois):
    B = features.shape[0]
    table = jnp.transpose(features, (0, 2, 3, 1)).reshape(B * H * W, C)
    rpad = jnp.zeros((NPAD, 8), jnp.float32).at[:NROI, :5].set(rois)
    mesh = plsc.VectorSubcoreMesh(core_axis_name="c", subcore_axis_name="s")
    run = functools.partial(
        pl.kernel,
        mesh=mesh,
        out_type=jax.ShapeDtypeStruct((NPAD, OUT * OUT, C), jnp.float32),
        scratch_types=[
            pltpu.VMEM((PER_W, 8), jnp.float32),
            pltpu.VMEM((4, 16), jnp.int32),
            pltpu.VMEM((4, 16), jnp.float32),
            pltpu.VMEM((8, 16), jnp.float32),
            pltpu.VMEM((128,), jnp.int32),
            pltpu.VMEM((128, C), jnp.float32),
            pltpu.VMEM((OUT * OUT, C), jnp.float32),
            pltpu.SemaphoreType.DMA,
        ],
    )(_body)
    out = run(table, rpad)
    out = out[:NROI].reshape(NROI, OUT, OUT, C).transpose(0, 3, 1, 2)
    return out


# speedup vs baseline: 25.3575x; 25.3575x over previous
import functools

import jax
import jax.numpy as jnp
from jax import lax
from jax.experimental import pallas as pl
from jax.experimental.pallas import tpu as pltpu
from jax.experimental.pallas import tpu_sc as plsc

H, W, C = 200, 304, 256
OUT = 7
SCALE = 0.25
SNUM = 2
NPTS = OUT * SNUM
NROI = 5000
NW = 32
PER_W = 157
NPAD = NW * PER_W


def _bcast(vec, i):
    idx = jnp.full((16, 1), i, dtype=jnp.int32)
    dnums = lax.GatherDimensionNumbers(
        offset_dims=(), collapsed_slice_dims=(0,), start_index_map=(0,))
    return lax.gather(vec, idx, dnums, (1,),
                      mode=lax.GatherScatterMode.PROMISE_IN_BOUNDS)


def _body(table, rois, out, roi_v, idxb, rows, obuf, sem):
    wid = lax.axis_index("s") * 2 + lax.axis_index("c")
    base = wid * PER_W
    pltpu.sync_copy(rois.at[pl.ds(base * 16, PER_W * 16)], roi_v)

    lane = lax.iota(jnp.int32, 16)
    pv = lax.shift_right_logical(lane, 1).astype(jnp.float32)
    iv = (lane & 1).astype(jnp.float32)
    lanemask = lane < NPTS

    def roi_body(n, carry):
        rv = roi_v[pl.ds(n * 16, 16)]
        bb = _bcast(rv, 0).astype(jnp.int32) * (H * W)
        sw = _bcast(rv, 1) * SCALE
        sh = _bcast(rv, 2) * SCALE
        ew = _bcast(rv, 3) * SCALE
        eh = _bcast(rv, 4) * SCALE
        bw = jnp.maximum(ew - sw, 1.0) * (1.0 / OUT)
        bh = jnp.maximum(eh - sh, 1.0) * (1.0 / OUT)
        xs = sw + pv * bw + (iv + 0.5) * (bw * (1.0 / SNUM))
        ys = sh + pv * bh + (iv + 0.5) * (bh * (1.0 / SNUM))
        vx = (xs > -1.0) & (xs < float(W)) & lanemask
        vy = (ys > -1.0) & (ys < float(H)) & lanemask
        xc = jnp.maximum(xs, 0.0)
        yc = jnp.maximum(ys, 0.0)
        x0 = jnp.minimum(xc.astype(jnp.int32), W - 1)
        y0 = jnp.minimum(yc.astype(jnp.int32), H - 1)
        x1 = jnp.minimum(x0 + 1, W - 1)
        y1 = jnp.minimum(y0 + 1, H - 1)
        lx = xc - x0.astype(jnp.float32)
        ly = yc - y0.astype(jnp.float32)
        zero = jnp.zeros_like(lx)
        hxv = jnp.where(vx, (1.0 - lx) * 0.5, zero)
        lxv = jnp.where(vx, lx * 0.5, zero)
        hyv = jnp.where(vy, (1.0 - ly) * 0.5, zero)
        lyv = jnp.where(vy, ly * 0.5, zero)
        rb0 = bb + y0 * W
        rb1 = bb + y1 * W

        for py in range(OUT):
            wrow = [None] * 8
            for si in range(2):
                i = py * 2 + si
                for a in range(2):
                    rb = _bcast(rb0 if a == 0 else rb1, i)
                    wy = _bcast(hyv if a == 0 else lyv, i)
                    g = (si * 2 + a) * 2
                    idxb[pl.ds(g * 16, 16)] = rb + x0
                    idxb[pl.ds((g + 1) * 16, 16)] = rb + x1
                    wrow[g] = wy * hxv
                    wrow[g + 1] = wy * lxv
            copies = [
                pltpu.async_copy(table.at[idxb.at[pl.ds(g * 16, 16)]],
                                 rows.at[pl.ds(g * 16, 16)], sem)
                for g in range(8)
            ]
            for c in copies:
                c.wait()

            for px in range(OUT):
                wv = []
                rrow = []
                for g in range(8):
                    for sj in range(2):
                        j = px * 2 + sj
                        wv.append(_bcast(wrow[g], j))
                        rrow.append(g * 16 + j)

                def cc_body(cc, _, wv=wv, rrow=rrow, px=px, py=py):
                    co = cc * 16
                    acc = wv[0] * rows[rrow[0], pl.ds(co, 16)]
                    for t in range(1, 16):
                        acc = acc + wv[t] * rows[rrow[t], pl.ds(co, 16)]
                    obuf[pl.ds((py * OUT + px) * C + co, 16)] = acc
                    return 0

                lax.fori_loop(0, C // 16, cc_body, 0)

        pltpu.sync_copy(obuf, out.at[base + n])
        return carry

    lax.fori_loop(0, PER_W, roi_body, 0)


def kernel(features, rois):
    B = features.shape[0]
    table = jnp.transpose(features, (0, 2, 3, 1)).reshape(B * H * W, C)
    rpad = jnp.zeros((NPAD, 16), jnp.float32).at[:NROI, :5].set(rois)
    mesh = plsc.VectorSubcoreMesh(core_axis_name="c", subcore_axis_name="s")
    run = functools.partial(
        pl.kernel,
        mesh=mesh,
        out_type=jax.ShapeDtypeStruct((NPAD, OUT * OUT * C), jnp.float32),
        scratch_types=[
            pltpu.VMEM((PER_W * 16,), jnp.float32),
            pltpu.VMEM((128,), jnp.int32),
            pltpu.VMEM((128, C), jnp.float32),
            pltpu.VMEM((OUT * OUT * C,), jnp.float32),
            pltpu.SemaphoreType.DMA,
        ],
    )(_body)
    out = run(table, rpad.reshape(NPAD * 16))
    out = out[:NROI].reshape(NROI, OUT, OUT, C).transpose(0, 3, 1, 2)
    return out


# speedup vs baseline: 36.6209x; 1.4442x over previous
import functools

import jax
import jax.numpy as jnp
from jax import lax
from jax.experimental import pallas as pl
from jax.experimental.pallas import tpu as pltpu
from jax.experimental.pallas import tpu_sc as plsc

H, W, C = 200, 304, 256
OUT = 7
SCALE = 0.25
SNUM = 2
NPTS = OUT * SNUM
NROI = 5000
NW = 32
PER_W = 157
NPAD = NW * PER_W


def _bcast(vec, i):
    idx = jnp.full((16, 1), i, dtype=jnp.int32)
    dnums = lax.GatherDimensionNumbers(
        offset_dims=(), collapsed_slice_dims=(0,), start_index_map=(0,))
    return lax.gather(vec, idx, dnums, (1,),
                      mode=lax.GatherScatterMode.PROMISE_IN_BOUNDS)


def _body(table, rois, out, roi_v, idxb, rows, obuf, sem):
    wid = lax.axis_index("s") * 2 + lax.axis_index("c")
    base = wid * PER_W
    pltpu.sync_copy(rois.at[pl.ds(base * 16, PER_W * 16)], roi_v)

    lane = lax.iota(jnp.int32, 16)
    pv = lax.shift_right_logical(lane, 1).astype(jnp.float32)
    iv = (lane & 1).astype(jnp.float32)
    lanemask = lane < NPTS

    def roi_body(n, carry):
        rv = roi_v[pl.ds(n * 16, 16)]
        bb = _bcast(rv, 0).astype(jnp.int32) * (H * W)
        sw = _bcast(rv, 1) * SCALE
        sh = _bcast(rv, 2) * SCALE
        ew = _bcast(rv, 3) * SCALE
        eh = _bcast(rv, 4) * SCALE
        bw = jnp.maximum(ew - sw, 1.0) * (1.0 / OUT)
        bh = jnp.maximum(eh - sh, 1.0) * (1.0 / OUT)
        xs = sw + pv * bw + (iv + 0.5) * (bw * (1.0 / SNUM))
        ys = sh + pv * bh + (iv + 0.5) * (bh * (1.0 / SNUM))
        vx = (xs > -1.0) & (xs < float(W)) & lanemask
        vy = (ys > -1.0) & (ys < float(H)) & lanemask
        xc = jnp.maximum(xs, 0.0)
        yc = jnp.maximum(ys, 0.0)
        x0 = jnp.minimum(xc.astype(jnp.int32), W - 1)
        y0 = jnp.minimum(yc.astype(jnp.int32), H - 1)
        x1 = jnp.minimum(x0 + 1, W - 1)
        y1 = jnp.minimum(y0 + 1, H - 1)
        lx = xc - x0.astype(jnp.float32)
        ly = yc - y0.astype(jnp.float32)
        zero = jnp.zeros_like(lx)
        hxv = jnp.where(vx, (1.0 - lx) * 0.5, zero)
        lxv = jnp.where(vx, lx * 0.5, zero)
        hyv = jnp.where(vy, (1.0 - ly) * 0.5, zero)
        lyv = jnp.where(vy, ly * 0.5, zero)
        rb0 = bb + y0 * W
        rb1 = bb + y1 * W

        for py in range(OUT):
            for si in range(2):
                i = py * 2 + si
                for a in range(2):
                    rb = _bcast(rb0 if a == 0 else rb1, i)
                    g = (si * 2 + a) * 2
                    idxb[pl.ds((py * 8 + g) * 16, 16)] = rb + x0
                    idxb[pl.ds((py * 8 + g + 1) * 16, 16)] = rb + x1

        def issue(py, b):
            return [
                pltpu.async_copy(
                    table.at[idxb.at[pl.ds((py * 8 + g) * 16, 16)]],
                    rows.at[b, pl.ds(g * 16, 16)], sem)
                for g in range(8)
            ]

        pending = issue(0, 0)
        for py in range(OUT):
            buf_i = py & 1
            nxt = issue(py + 1, 1 - buf_i) if py < OUT - 1 else []
            for c in pending:
                c.wait()
            pending = nxt

            wrow = [None] * 8
            for si in range(2):
                i = py * 2 + si
                for a in range(2):
                    wy = _bcast(hyv if a == 0 else lyv, i)
                    g = (si * 2 + a) * 2
                    wrow[g] = wy * hxv
                    wrow[g + 1] = wy * lxv

            for px in range(OUT):
                wv = []
                rrow = []
                for g in range(8):
                    for sj in range(2):
                        j = px * 2 + sj
                        wv.append(_bcast(wrow[g], j))
                        rrow.append(g * 16 + j)

                def cc_body(cc, _, wv=wv, rrow=rrow, px=px, py=py,
                            buf_i=buf_i):
                    co = cc * 16
                    acc = wv[0] * rows[buf_i, rrow[0], pl.ds(co, 16)]
                    for t in range(1, 16):
                        acc = acc + wv[t] * rows[buf_i, rrow[t], pl.ds(co, 16)]
                    obuf[pl.ds((py * OUT + px) * C + co, 16)] = acc
                    return 0

                lax.fori_loop(0, C // 16, cc_body, 0)

        pltpu.sync_copy(obuf, out.at[base + n])
        return carry

    lax.fori_loop(0, PER_W, roi_body, 0)


def kernel(features, rois):
    B = features.shape[0]
    table = jnp.transpose(features, (0, 2, 3, 1)).reshape(B * H * W, C)
    rpad = jnp.zeros((NPAD, 16), jnp.float32).at[:NROI, :5].set(rois)
    mesh = plsc.VectorSubcoreMesh(core_axis_name="c", subcore_axis_name="s")
    run = functools.partial(
        pl.kernel,
        mesh=mesh,
        out_type=jax.ShapeDtypeStruct((NPAD, OUT * OUT * C), jnp.float32),
        scratch_types=[
            pltpu.VMEM((PER_W * 16,), jnp.float32),
            pltpu.VMEM((OUT * 128,), jnp.int32),
            pltpu.VMEM((2, 128, C), jnp.float32),
            pltpu.VMEM((OUT * OUT * C,), jnp.float32),
            pltpu.SemaphoreType.DMA,
        ],
    )(_body)
    out = run(table, rpad.reshape(NPAD * 16))
    out = out[:NROI].reshape(NROI, OUT, OUT, C).transpose(0, 3, 1, 2)
    return out


# speedup vs baseline: 39.8153x; 1.0872x over previous
import functools

import jax
import jax.numpy as jnp
from jax import lax
from jax.experimental import pallas as pl
from jax.experimental.pallas import tpu as pltpu
from jax.experimental.pallas import tpu_sc as plsc

H, W, C = 200, 304, 256
OUT = 7
SCALE = 0.25
SNUM = 2
NPTS = OUT * SNUM
NROI = 5000
NW = 32
PER_W = 157
NPAD = NW * PER_W


def _bcast(vec, i):
    idx = jnp.full((16, 1), i, dtype=jnp.int32)
    dnums = lax.GatherDimensionNumbers(
        offset_dims=(), collapsed_slice_dims=(0,), start_index_map=(0,))
    return lax.gather(vec, idx, dnums, (1,),
                      mode=lax.GatherScatterMode.PROMISE_IN_BOUNDS)


def _body(table, rois, out, roi_v, idxb, rows, obuf, sem):
    wid = lax.axis_index("s") * 2 + lax.axis_index("c")
    base = wid * PER_W
    pltpu.sync_copy(rois.at[pl.ds(base * 16, PER_W * 16)], roi_v)

    lane = lax.iota(jnp.int32, 16)
    pv = lax.shift_right_logical(lane, 1).astype(jnp.float32)
    iv = (lane & 1).astype(jnp.float32)
    lanemask = lane < NPTS

    def roi_body(n, carry):
        rv = roi_v[pl.ds(n * 16, 16)]
        bb = _bcast(rv, 0).astype(jnp.int32) * (H * W)
        sw = _bcast(rv, 1) * SCALE
        sh = _bcast(rv, 2) * SCALE
        ew = _bcast(rv, 3) * SCALE
        eh = _bcast(rv, 4) * SCALE
        bw = jnp.maximum(ew - sw, 1.0) * (1.0 / OUT)
        bh = jnp.maximum(eh - sh, 1.0) * (1.0 / OUT)
        xs = sw + pv * bw + (iv + 0.5) * (bw * (1.0 / SNUM))
        ys = sh + pv * bh + (iv + 0.5) * (bh * (1.0 / SNUM))
        vx = (xs > -1.0) & (xs < float(W)) & lanemask
        vy = (ys > -1.0) & (ys < float(H)) & lanemask
        xc = jnp.maximum(xs, 0.0)
        yc = jnp.maximum(ys, 0.0)
        x0 = jnp.minimum(xc.astype(jnp.int32), W - 1)
        y0 = jnp.minimum(yc.astype(jnp.int32), H - 1)
        x1 = jnp.minimum(x0 + 1, W - 1)
        y1 = jnp.minimum(y0 + 1, H - 1)
        lx = xc - x0.astype(jnp.float32)
        ly = yc - y0.astype(jnp.float32)
        zero = jnp.zeros_like(lx)
        hxv = jnp.where(vx, (1.0 - lx) * 0.5, zero)
        lxv = jnp.where(vx, lx * 0.5, zero)
        hyv = jnp.where(vy, (1.0 - ly) * 0.5, zero)
        lyv = jnp.where(vy, ly * 0.5, zero)
        rb0 = bb + y0 * W
        rb1 = bb + y1 * W

        for py in range(OUT):
            for si in range(2):
                i = py * 2 + si
                for a in range(2):
                    rb = _bcast(rb0 if a == 0 else rb1, i)
                    g = (si * 2 + a) * 2
                    idxb[pl.ds((py * 8 + g) * 16, 16)] = rb + x0
                    idxb[pl.ds((py * 8 + g + 1) * 16, 16)] = rb + x1

        def issue(py, b):
            return [
                pltpu.async_copy(
                    table.at[idxb.at[pl.ds((py * 8 + g) * 16, 16)]],
                    rows.at[b, pl.ds(g * 16, 16)], sem)
                for g in range(8)
            ]

        pending = issue(0, 0)
        for py in range(OUT):
            buf_i = py & 1
            nxt = issue(py + 1, 1 - buf_i) if py < OUT - 1 else []
            for c in pending:
                c.wait()
            pending = nxt

            wrow = [None] * 8
            for si in range(2):
                i = py * 2 + si
                for a in range(2):
                    wy = _bcast(hyv if a == 0 else lyv, i)
                    g = (si * 2 + a) * 2
                    wrow[g] = wy * hxv
                    wrow[g + 1] = wy * lxv

            for px in range(OUT):
                wv = []
                rrow = []
                for g in range(8):
                    for sj in range(2):
                        j = px * 2 + sj
                        wv.append(_bcast(wrow[g], j))
                        rrow.append(g * 16 + j)

                def cc_body(cc, _, wv=wv, rrow=rrow, px=px, py=py,
                            buf_i=buf_i):
                    co = cc * 32
                    c1 = co + 16
                    acc0 = wv[0] * rows[buf_i, rrow[0], pl.ds(co, 16)]
                    acc1 = wv[0] * rows[buf_i, rrow[0], pl.ds(c1, 16)]
                    for t in range(1, 16):
                        acc0 = acc0 + wv[t] * rows[buf_i, rrow[t],
                                                   pl.ds(co, 16)]
                        acc1 = acc1 + wv[t] * rows[buf_i, rrow[t],
                                                   pl.ds(c1, 16)]
                    obuf[pl.ds((py * OUT + px) * C + co, 16)] = acc0
                    obuf[pl.ds((py * OUT + px) * C + c1, 16)] = acc1
                    return 0

                lax.fori_loop(0, C // 32, cc_body, 0)

        pltpu.sync_copy(obuf, out.at[base + n])
        return carry

    lax.fori_loop(0, PER_W, roi_body, 0)


def kernel(features, rois):
    B = features.shape[0]
    table = jnp.transpose(features, (0, 2, 3, 1)).reshape(B * H * W, C)
    rpad = jnp.zeros((NPAD, 16), jnp.float32).at[:NROI, :5].set(rois)
    mesh = plsc.VectorSubcoreMesh(core_axis_name="c", subcore_axis_name="s")
    run = functools.partial(
        pl.kernel,
        mesh=mesh,
        out_type=jax.ShapeDtypeStruct((NPAD, OUT * OUT * C), jnp.float32),
        scratch_types=[
            pltpu.VMEM((PER_W * 16,), jnp.float32),
            pltpu.VMEM((OUT * 128,), jnp.int32),
            pltpu.VMEM((2, 128, C), jnp.float32),
            pltpu.VMEM((OUT * OUT * C,), jnp.float32),
            pltpu.SemaphoreType.DMA,
        ],
    )(_body)
    out = run(table, rpad.reshape(NPAD * 16))
    out = out[:NROI].reshape(NROI, OUT, OUT, C).transpose(0, 3, 1, 2)
    return out
